# Initial kernel scaffold; baseline (speedup 1.0000x reference)
import functools
import math

import jax
import jax.numpy as jnp
from jax.experimental import pallas as pl

H1, D1 = 10, 16
H2, D2 = 10, 32


def _gelu(x):
    c = math.sqrt(2.0 / math.pi)
    return 0.5 * x * (1.0 + jnp.tanh(c * (x + 0.044715 * (x * x * x))))


def _elu(x):
    return jnp.where(x > 0, x, jnp.expm1(x))


def _ident(x):
    return x


def _mm_body(x_ref, w_ref, b_ref, o_ref, *, act):
    acc = jnp.dot(x_ref[...], w_ref[...], preferred_element_type=jnp.float32)
    acc = acc + b_ref[...]
    o_ref[...] = act(acc)


def _mm(x, w, b=None, act=_ident, bm=1024):
    m, k = x.shape
    n = w.shape[1]
    if b is None:
        b = jnp.zeros((n,), dtype=jnp.float32)
    b2 = b.reshape(1, n)
    grid = (pl.cdiv(m, bm),)
    return pl.pallas_call(
        functools.partial(_mm_body, act=act),
        grid=grid,
        in_specs=[
            pl.BlockSpec((bm, k), lambda i: (i, 0)),
            pl.BlockSpec((k, n), lambda i: (0, 0)),
            pl.BlockSpec((1, n), lambda i: (0, 0)),
        ],
        out_specs=pl.BlockSpec((bm, n), lambda i: (i, 0)),
        out_shape=jax.ShapeDtypeStruct((m, n), jnp.float32),
    )(x, w, b2)


def _segment_softmax_aggregate(logits, dst, v_src, n_dst, h, d):
    w = jnp.exp(logits)
    s = jax.ops.segment_sum(w, dst, num_segments=n_dst)
    msg = (v_src.reshape(-1, h, d) * w[:, :, None]).reshape(-1, h * d)
    agg = jax.ops.segment_sum(msg, dst, num_segments=n_dst)
    agg = agg.reshape(-1, h, d) / (s[:, :, None] + 1e-16)
    return agg.reshape(-1, h * d)


def _het_conv(xd, eid, ead, p, h, d):
    k = {t: _mm(xd[t], p['Wk_' + t]) for t in xd}
    q = {t: _mm(xd[t], p['Wq_' + t]) for t in xd}
    v = {t: _mm(xd[t], p['Wv_' + t]) for t in xd}
    agg = {t: jnp.zeros((xd[t].shape[0], h * d), dtype=jnp.float32) for t in xd}
    for et in eid:
        src_t, dst_t = et.split('__')
        src, dst = eid[et][0], eid[et][1]
        n_dst = xd[dst_t].shape[0]
        eb = ead[et] @ p['We_' + et]
        ks = k[src_t][src].reshape(-1, h, d)
        qd = q[dst_t][dst].reshape(-1, h, d)
        logits = (ks * qd).sum(-1) / math.sqrt(float(d)) + eb
        agg[dst_t] = agg[dst_t] + _segment_softmax_aggregate(
            logits, dst, v[src_t][src], n_dst, h, d)
    return {t: _mm(_gelu(agg[t]), p['Wa_' + t]) for t in xd}


def _bn(x, g, b):
    mu = x.mean(axis=0)
    var = x.var(axis=0)
    return (x - mu) / jnp.sqrt(var + 1e-5) * g + b


def kernel(x_ant, x_user, edge_index_ant_user, edge_index_user_ant,
           edge_attr_ant_user, edge_attr_user_ant, params):
    xd = {'ant': x_ant, 'user': x_user}
    eid = {'ant__user': edge_index_ant_user, 'user__ant': edge_index_user_ant}
    ead = {'ant__user': edge_attr_ant_user, 'user__ant': edge_attr_user_ant}
    p = params
    xd = _het_conv(xd, eid, ead, p['conv1'], H1, D1)
    xd = {t: _elu(xd[t]) for t in xd}
    xd = _het_conv(xd, eid, ead, p['conv2'], H2, D2)
    xd = {t: _elu(xd[t]) for t in xd}
    xd = {t: _mm(xd[t], p['lin1_' + t + '_W'], p['lin1_' + t + '_b'], act=_elu)
          for t in xd}
    xd = {t: _bn(xd[t], p['bn_' + t + '_g'], p['bn_' + t + '_b']) for t in xd}
    xd = {t: _mm(xd[t], p['lin2_' + t + '_W'], p['lin2_' + t + '_b'])
          for t in xd}
    out = {t: jax.nn.softmax(xd[t], axis=-1) for t in xd}
    return (out['ant'], out['user'])


# speedup vs baseline: 10.1738x; 10.1738x over previous
import functools
import math

import jax
import jax.numpy as jnp
from jax.experimental import pallas as pl

H1, D1 = 10, 16
H2, D2 = 10, 32


def _gelu(x):
    c = math.sqrt(2.0 / math.pi)
    return 0.5 * x * (1.0 + jnp.tanh(c * (x + 0.044715 * (x * x * x))))


def _elu(x):
    return jnp.where(x > 0, x, jnp.exp(jnp.minimum(x, 0.0)) - 1.0)


def _ident(x):
    return x


def _mm_body(x_ref, w_ref, b_ref, o_ref, *, act):
    acc = jnp.dot(x_ref[...], w_ref[...], preferred_element_type=jnp.float32)
    acc = acc + b_ref[...]
    o_ref[...] = act(acc)


def _mm(x, w, b=None, act=_ident, bm=1024):
    m, k = x.shape
    n = w.shape[1]
    if b is None:
        b = jnp.zeros((n,), dtype=jnp.float32)
    b2 = b.reshape(1, n)
    grid = (pl.cdiv(m, bm),)
    return pl.pallas_call(
        functools.partial(_mm_body, act=act),
        grid=grid,
        in_specs=[
            pl.BlockSpec((bm, k), lambda i: (i, 0)),
            pl.BlockSpec((k, n), lambda i: (0, 0)),
            pl.BlockSpec((1, n), lambda i: (0, 0)),
        ],
        out_specs=pl.BlockSpec((bm, n), lambda i: (i, 0)),
        out_shape=jax.ShapeDtypeStruct((m, n), jnp.float32),
    )(x, w, b2)


def _segment_softmax_aggregate(logits, dst, v_src, n_dst, h, d):
    w = jnp.exp(logits)
    s = jax.ops.segment_sum(w, dst, num_segments=n_dst)
    msg = (v_src.reshape(-1, h, d) * w[:, :, None]).reshape(-1, h * d)
    agg = jax.ops.segment_sum(msg, dst, num_segments=n_dst)
    agg = agg.reshape(-1, h, d) / (s[:, :, None] + 1e-16)
    return agg.reshape(-1, h * d)


def _het_conv(xd, eid, ead, p, h, d):
    k = {t: _mm(xd[t], p['Wk_' + t]) for t in xd}
    q = {t: _mm(xd[t], p['Wq_' + t]) for t in xd}
    v = {t: _mm(xd[t], p['Wv_' + t]) for t in xd}
    agg = {t: jnp.zeros((xd[t].shape[0], h * d), dtype=jnp.float32) for t in xd}
    for et in eid:
        src_t, dst_t = et.split('__')
        src, dst = eid[et][0], eid[et][1]
        n_dst = xd[dst_t].shape[0]
        eb = ead[et] @ p['We_' + et]
        ks = k[src_t][src].reshape(-1, h, d)
        qd = q[dst_t][dst].reshape(-1, h, d)
        logits = (ks * qd).sum(-1) / math.sqrt(float(d)) + eb
        agg[dst_t] = agg[dst_t] + _segment_softmax_aggregate(
            logits, dst, v[src_t][src], n_dst, h, d)
    return {t: _mm(_gelu(agg[t]), p['Wa_' + t]) for t in xd}


def _bn(x, g, b):
    mu = x.mean(axis=0)
    var = x.var(axis=0)
    return (x - mu) / jnp.sqrt(var + 1e-5) * g + b


def kernel(x_ant, x_user, edge_index_ant_user, edge_index_user_ant,
           edge_attr_ant_user, edge_attr_user_ant, params):
    xd = {'ant': x_ant, 'user': x_user}
    eid = {'ant__user': edge_index_ant_user, 'user__ant': edge_index_user_ant}
    ead = {'ant__user': edge_attr_ant_user, 'user__ant': edge_attr_user_ant}
    p = params
    xd = _het_conv(xd, eid, ead, p['conv1'], H1, D1)
    xd = {t: _elu(xd[t]) for t in xd}
    xd = _het_conv(xd, eid, ead, p['conv2'], H2, D2)
    xd = {t: _elu(xd[t]) for t in xd}
    xd = {t: _mm(xd[t], p['lin1_' + t + '_W'], p['lin1_' + t + '_b'], act=_elu)
          for t in xd}
    xd = {t: _bn(xd[t], p['bn_' + t + '_g'], p['bn_' + t + '_b']) for t in xd}
    xd = {t: _mm(xd[t], p['lin2_' + t + '_W'], p['lin2_' + t + '_b'])
          for t in xd}
    out = {t: jax.nn.softmax(xd[t], axis=-1) for t in xd}
    return (out['ant'], out['user'])
